# baseline (speedup 1.0000x reference)
import functools
import math

import jax
import jax.numpy as jnp
from jax import lax
from jax.experimental import pallas as pl
from jax.experimental.pallas import tpu as pltpu
from jax.experimental.pallas import tpu_sc as plsc

D_MODEL = 64
SCALE = math.sqrt(D_MODEL)

_NC = 2
_NS = 16
_NW = _NC * _NS
_CHUNK = 128


def _make_sc_lookup(n_rows):
    assert n_rows % (_NW * _CHUNK) == 0
    chunks_per_w = n_rows // (_NW * _CHUNK)
    mesh = plsc.VectorSubcoreMesh(core_axis_name="c", subcore_axis_name="s")

    def body(idx_hbm, table_hbm, out_hbm, idx_v, rows_v, sem):
        wid = lax.axis_index("s") * _NC + lax.axis_index("c")
        chunk_base = wid * chunks_per_w
        pltpu.sync_copy(idx_hbm.at[pl.ds(chunk_base, chunks_per_w)], idx_v)

        def do_chunk(g, carry):
            pltpu.async_copy(table_hbm.at[idx_v.at[g]], rows_v, sem).wait()

            def scale_row(r, c2):
                for d in range(D_MODEL // 16):
                    sl = (r, pl.ds(d * 16, 16))
                    rows_v[sl] = rows_v[sl] * SCALE
                return c2

            lax.fori_loop(0, _CHUNK, scale_row, 0)
            pltpu.sync_copy(
                rows_v, out_hbm.at[pl.ds((chunk_base + g) * _CHUNK, _CHUNK)]
            )
            return carry

        lax.fori_loop(0, chunks_per_w, do_chunk, 0)

    return pl.kernel(
        body,
        out_type=jax.ShapeDtypeStruct((n_rows, D_MODEL), jnp.float32),
        mesh=mesh,
        compiler_params=pltpu.CompilerParams(use_tc_tiling_on_sc=False),
        scratch_types=[
            pltpu.VMEM((chunks_per_w, _CHUNK), jnp.int32),
            pltpu.VMEM((_CHUNK, D_MODEL), jnp.float32),
            pltpu.SemaphoreType.DMA,
        ],
    )


def kernel(x, embedding):
    n_rows = x.size
    idx = x.reshape(n_rows // _CHUNK, _CHUNK).astype(jnp.int32)
    out = _make_sc_lookup(n_rows)(idx, embedding)
    return out.reshape(x.shape + (D_MODEL,))


# speedup vs baseline: 1.2060x; 1.2060x over previous
import math

import jax
import jax.numpy as jnp
from jax import lax
from jax.experimental import pallas as pl
from jax.experimental.pallas import tpu as pltpu
from jax.experimental.pallas import tpu_sc as plsc

D_MODEL = 64
SCALE = math.sqrt(D_MODEL)

_NC = 2
_NS = 16
_NW = _NC * _NS
_CHUNK = 128
_NBUF = 4


def _make_sc_lookup(n_rows):
    assert n_rows % (_NW * _CHUNK * _NBUF) == 0
    chunks_per_w = n_rows // (_NW * _CHUNK)
    n_super = chunks_per_w // _NBUF
    mesh = plsc.VectorSubcoreMesh(core_axis_name="c", subcore_axis_name="s")

    def body(idx_hbm, table_hbm, out_hbm, idx_v, gbufs, sbufs, gsems, ssems):
        wid = lax.axis_index("s") * _NC + lax.axis_index("c")
        chunk_base = wid * chunks_per_w
        pltpu.sync_copy(idx_hbm.at[pl.ds(chunk_base, chunks_per_w)], idx_v)

        def start_gather(b, g):
            pltpu.async_copy(table_hbm.at[idx_v.at[g]], gbufs[b], gsems[b])

        def wait_gather(b, g):
            pltpu.make_async_copy(
                table_hbm.at[idx_v.at[g]], gbufs[b], gsems[b]
            ).wait()

        def out_slice(g):
            return out_hbm.at[pl.ds((chunk_base + g) * _CHUNK, _CHUNK)]

        def start_store(b, g):
            pltpu.async_copy(sbufs[b], out_slice(g), ssems[b])

        def wait_store(b, g):
            pltpu.make_async_copy(sbufs[b], out_slice(g), ssems[b]).wait()

        def scale(b):
            def scale_row(r, c):
                for d in range(D_MODEL // 16):
                    sl = (r, pl.ds(d * 16, 16))
                    sbufs[b][sl] = gbufs[b][sl] * SCALE
                return c

            lax.fori_loop(0, _CHUNK, scale_row, 0)

        for b in range(_NBUF):
            start_gather(b, b)

        for b in range(_NBUF):
            wait_gather(b, b)
            scale(b)
            start_store(b, b)
            start_gather(b, b + _NBUF)

        def super_it(s, carry):
            for b in range(_NBUF):
                g = s * _NBUF + b
                wait_gather(b, g)
                wait_store(b, g - _NBUF)
                scale(b)
                start_store(b, g)
                start_gather(b, g + _NBUF)
            return carry

        if n_super > 2:
            lax.fori_loop(1, n_super - 1, super_it, 0)

        for b in range(_NBUF):
            g = (n_super - 1) * _NBUF + b
            wait_gather(b, g)
            wait_store(b, g - _NBUF)
            scale(b)
            start_store(b, g)
        for b in range(_NBUF):
            g = (n_super - 1) * _NBUF + b
            wait_store(b, g)

    return pl.kernel(
        body,
        out_type=jax.ShapeDtypeStruct((n_rows, D_MODEL), jnp.float32),
        mesh=mesh,
        compiler_params=pltpu.CompilerParams(use_tc_tiling_on_sc=False),
        scratch_types=[
            pltpu.VMEM((chunks_per_w, _CHUNK), jnp.int32),
            [pltpu.VMEM((_CHUNK, D_MODEL), jnp.float32) for _ in range(_NBUF)],
            [pltpu.VMEM((_CHUNK, D_MODEL), jnp.float32) for _ in range(_NBUF)],
            [pltpu.SemaphoreType.DMA for _ in range(_NBUF)],
            [pltpu.SemaphoreType.DMA for _ in range(_NBUF)],
        ],
    )


def kernel(x, embedding):
    n_rows = x.size
    idx = x.reshape(n_rows // _CHUNK, _CHUNK).astype(jnp.int32)
    out = _make_sc_lookup(n_rows)(idx, embedding)
    return out.reshape(x.shape + (D_MODEL,))


# speedup vs baseline: 1.2845x; 1.0651x over previous
import math

import jax
import jax.numpy as jnp
from jax import lax
from jax.experimental import pallas as pl
from jax.experimental.pallas import tpu as pltpu
from jax.experimental.pallas import tpu_sc as plsc

D_MODEL = 64
SCALE = math.sqrt(D_MODEL)

_NC = 2
_NS = 16
_NW = _NC * _NS
_CHUNK = 128
_NBUF = 2


def _make_sc_lookup(n_rows):
    assert n_rows % (_NW * _CHUNK * _NBUF) == 0
    chunks_per_w = n_rows // (_NW * _CHUNK)
    n_super = chunks_per_w // _NBUF
    mesh = plsc.VectorSubcoreMesh(core_axis_name="c", subcore_axis_name="s")

    def body(idx_hbm, table_hbm, out_hbm, idx_v, gbufs, sbufs, gsems, ssems):
        wid = lax.axis_index("s") * _NC + lax.axis_index("c")
        chunk_base = wid * chunks_per_w
        pltpu.sync_copy(idx_hbm.at[pl.ds(chunk_base, chunks_per_w)], idx_v)

        def start_gather(b, g):
            pltpu.async_copy(table_hbm.at[idx_v.at[g]], gbufs[b], gsems[b])

        def wait_gather(b, g):
            pltpu.make_async_copy(
                table_hbm.at[idx_v.at[g]], gbufs[b], gsems[b]
            ).wait()

        def out_slice(g):
            return out_hbm.at[pl.ds((chunk_base + g) * _CHUNK, _CHUNK)]

        def start_store(b, g):
            pltpu.async_copy(sbufs[b], out_slice(g), ssems[b])

        def wait_store(b, g):
            pltpu.make_async_copy(sbufs[b], out_slice(g), ssems[b]).wait()

        def scale(b):
            def scale_row(r, c):
                for d in range(D_MODEL // 16):
                    sl = pl.ds(d * 16, 16)
                    sbufs[b][r, sl] = gbufs[b][r, sl] * SCALE
                return c

            lax.fori_loop(0, _CHUNK, scale_row, 0)

        for b in range(_NBUF):
            start_gather(b, b)

        for b in range(_NBUF):
            wait_gather(b, b)
            scale(b)
            start_store(b, b)
            start_gather(b, b + _NBUF)

        def super_it(s, carry):
            for b in range(_NBUF):
                g = s * _NBUF + b
                wait_gather(b, g)
                wait_store(b, g - _NBUF)
                scale(b)
                start_store(b, g)
                start_gather(b, g + _NBUF)
            return carry

        if n_super > 2:
            lax.fori_loop(1, n_super - 1, super_it, 0)

        for b in range(_NBUF):
            g = (n_super - 1) * _NBUF + b
            wait_gather(b, g)
            wait_store(b, g - _NBUF)
            scale(b)
            start_store(b, g)
        for b in range(_NBUF):
            g = (n_super - 1) * _NBUF + b
            wait_store(b, g)

    return pl.kernel(
        body,
        out_type=jax.ShapeDtypeStruct((n_rows, D_MODEL), jnp.float32),
        mesh=mesh,
        scratch_types=[
            pltpu.VMEM((chunks_per_w, _CHUNK), jnp.int32),
            [pltpu.VMEM((_CHUNK, 2 * D_MODEL), jnp.float32) for _ in range(_NBUF)],
            [pltpu.VMEM((_CHUNK, D_MODEL), jnp.float32) for _ in range(_NBUF)],
            [pltpu.SemaphoreType.DMA for _ in range(_NBUF)],
            [pltpu.SemaphoreType.DMA for _ in range(_NBUF)],
        ],
    )


def kernel(x, embedding):
    n_rows = x.size
    idx = x.reshape(n_rows // _CHUNK, _CHUNK).astype(jnp.int32)
    t2d = jnp.concatenate([embedding, embedding], axis=1)
    out = _make_sc_lookup(n_rows)(idx, t2d)
    return out.reshape(x.shape + (D_MODEL,))
